# Initial kernel scaffold; baseline (speedup 1.0000x reference)
import jax
import jax.numpy as jnp
from jax.experimental import pallas as pl


def kernel(x, mask):
    raise NotImplementedError("write your pallas kernel here")


# speedup vs baseline: 1.1366x; 1.1366x over previous
import jax
import jax.numpy as jnp
from jax import lax
from jax.experimental import pallas as pl
from jax.experimental.pallas import tpu as pltpu, tpu_sc as plsc

N_ROWS, N_COLS = 1024, 32768
NUM_WORKERS = 32
ROWS_PER_W = N_ROWS // NUM_WORKERS
R = 16
W = 2048
NCT = N_COLS // W
L = 16


def _sc_body(x_hbm, m_hbm, out_hbm, xv, mv, ov, csm):
    wid = lax.axis_index("s") * 2 + lax.axis_index("c")

    for rg in range(ROWS_PER_W // R):
        r0 = wid * ROWS_PER_W + rg * R
        for r in range(R):
            csm[r] = 0.0

        def col_tile(t, carry_dummy):
            c0 = t * W
            pltpu.sync_copy(x_hbm.at[pl.ds(r0, R), pl.ds(c0, W)], xv)
            pltpu.sync_copy(m_hbm.at[pl.ds(r0, R), pl.ds(c0, W)], mv)

            def row_body(r, dummy):
                def vec_body(v, carry):
                    sl = pl.ds(v * L, L)
                    xm = xv[r, sl] * mv[r, sl]
                    s = plsc.cumsum(xm)
                    ov[r, sl] = s + jnp.broadcast_to(carry, (L,))
                    return carry + jnp.sum(xm)

                csm[r] = lax.fori_loop(0, W // L, vec_body, csm[r], unroll=4)
                return dummy

            lax.fori_loop(0, R, row_body, 0)
            pltpu.sync_copy(ov, out_hbm.at[pl.ds(r0, R), pl.ds(c0, W)])
            return carry_dummy

        lax.fori_loop(0, NCT, col_tile, 0)


def kernel(x, mask):
    m = mask.astype(jnp.float32)
    mesh = plsc.VectorSubcoreMesh(core_axis_name="c", subcore_axis_name="s")
    f = pl.kernel(
        _sc_body,
        out_type=jax.ShapeDtypeStruct((N_ROWS, N_COLS), jnp.float32),
        mesh=mesh,
        compiler_params=pltpu.CompilerParams(needs_layout_passes=False),
        scratch_types=[
            pltpu.VMEM((R, W), jnp.float32),
            pltpu.VMEM((R, W), jnp.float32),
            pltpu.VMEM((R, W), jnp.float32),
            pltpu.SMEM((R,), jnp.float32),
        ],
    )
    return f(x, m)


# speedup vs baseline: 1.9597x; 1.7242x over previous
import jax
import jax.numpy as jnp
from jax import lax
from jax.experimental import pallas as pl
from jax.experimental.pallas import tpu as pltpu, tpu_sc as plsc

N_ROWS, N_COLS = 1024, 32768
NUM_WORKERS = 32
ROWS_PER_W = N_ROWS // NUM_WORKERS
R = 16
W = 2048
NCT = N_COLS // W
L = 16
IL = 4


def _sc_body(x_hbm, m_hbm, out_hbm, xv, mv, ov, cv):
    wid = lax.axis_index("s") * 2 + lax.axis_index("c")
    idx_last = jnp.full((L, 1), L - 1, dtype=jnp.int32)
    dnums = lax.GatherDimensionNumbers(
        offset_dims=(), collapsed_slice_dims=(0,), start_index_map=(0,))

    def bcast_last(o):
        return lax.gather(o, idx_last, dnums, slice_sizes=(1,),
                          mode=lax.GatherScatterMode.PROMISE_IN_BOUNDS)
    zeros = jnp.zeros((L,), dtype=jnp.float32)

    for rg in range(ROWS_PER_W // R):
        r0 = wid * ROWS_PER_W + rg * R
        for r in range(R):
            cv[r, :] = zeros

        def col_tile(t, carry_dummy):
            c0 = t * W
            pltpu.sync_copy(x_hbm.at[pl.ds(r0, R), pl.ds(c0, W)], xv)
            pltpu.sync_copy(m_hbm.at[pl.ds(r0, R), pl.ds(c0, W)], mv)

            for rb in range(0, R, IL):
                def vec_body(v, carries):
                    new = []
                    for i in range(IL):
                        r = rb + i
                        sl = pl.ds(v * L, L)
                        xm = xv[r, sl] * mv[r, sl]
                        o = plsc.cumsum(xm) + carries[i]
                        ov[r, sl] = o
                        new.append(bcast_last(o))
                    return tuple(new)

                carries = lax.fori_loop(
                    0, W // L, vec_body,
                    tuple(cv[rb + i, :] for i in range(IL)), unroll=2)
                for i in range(IL):
                    cv[rb + i, :] = carries[i]

            pltpu.sync_copy(ov, out_hbm.at[pl.ds(r0, R), pl.ds(c0, W)])
            return carry_dummy

        lax.fori_loop(0, NCT, col_tile, 0)


def kernel(x, mask):
    m = mask.astype(jnp.float32)
    mesh = plsc.VectorSubcoreMesh(core_axis_name="c", subcore_axis_name="s")
    f = pl.kernel(
        _sc_body,
        out_type=jax.ShapeDtypeStruct((N_ROWS, N_COLS), jnp.float32),
        mesh=mesh,
        compiler_params=pltpu.CompilerParams(needs_layout_passes=False),
        scratch_types=[
            pltpu.VMEM((R, W), jnp.float32),
            pltpu.VMEM((R, W), jnp.float32),
            pltpu.VMEM((R, W), jnp.float32),
            pltpu.VMEM((R, L), jnp.float32),
        ],
    )
    return f(x, m)
